# baseline (speedup 1.0000x reference)
import functools

import jax
import jax.numpy as jnp
from jax.experimental import pallas as pl

_TOPK_FRAC = 0.1


def _psum(m, part_dtype):
    rows, n = m.shape
    mm = m.astype(part_dtype)
    if n % 128 == 0 and n > 128:
        acc = None
        while mm.shape[1] > 128:
            cols = mm.shape[1]
            if (cols // 128) % 2:
                tail = mm[:, cols - 128 :]
                acc = tail if acc is None else acc + tail
                mm = mm[:, : cols - 128]
            else:
                half = cols // 2
                mm = mm[:, :half] + mm[:, half:]
        if acc is not None:
            mm = mm + acc
    return jnp.sum(mm.astype(jnp.int32), axis=1, keepdims=True)


def _value_search(hi, lo, want):
    shape = want.shape

    def it_a(i, p):
        cand_u = p | ((1 << 15) >> i)
        cand = cand_u.astype(jnp.int16) ^ jnp.int16(-0x8000)
        ge = _psum(hi >= cand, jnp.int16)
        return jnp.where(ge >= want, cand_u, p)

    p_hi = jax.lax.fori_loop(0, 16, it_a, jnp.zeros(shape, jnp.int32))
    p_hi16 = p_hi.astype(jnp.int16) ^ jnp.int16(-0x8000)
    g1 = _psum(hi > p_hi16, jnp.int16)

    alo = jnp.where(hi == p_hi16, lo, jnp.int16(-1))
    want2 = want - g1

    def it_b(i, p):
        cand = (p | ((1 << 14) >> i)).astype(jnp.int16)
        ge = _psum(alo >= cand, jnp.int16)
        return jnp.where(ge >= want2, cand.astype(jnp.int32), p)

    p_lo = jax.lax.fori_loop(0, 15, it_b, jnp.zeros(shape, jnp.int32))
    p_lo16 = p_lo.astype(jnp.int16)
    g2 = _psum(alo > p_lo16, jnp.int16)
    ge2 = _psum(alo >= p_lo16, jnp.int16)

    vstar = (p_hi << 15) | p_lo
    return vstar, g1 + g2, g1 + ge2


def _fold_min(m):
    acc = None
    while m.shape[1] > 128:
        cols = m.shape[1]
        if (cols // 128) % 2 and cols > 256:
            tail = m[:, cols - 128 :]
            acc = tail if acc is None else jnp.minimum(acc, tail)
            m = m[:, : cols - 128]
        else:
            half = cols // 2
            m = jnp.minimum(m[:, :half], m[:, half:])
    if acc is not None:
        m = jnp.minimum(m, acc)
    return jnp.min(m, axis=1, keepdims=True)


def _index_search(midx, want):
    shape = want.shape

    def it(i, p):
        cand = p | ((1 << 15) >> i)
        c = _psum(midx < cand.astype(jnp.float32), jnp.float32)
        return jnp.where(c < want, cand, p)

    return jax.lax.fori_loop(0, 16, it, jnp.zeros(shape, jnp.int32))


def _body(x_ref, o_ref, *, k, n_cols):
    xv = x_ref[...]
    rows = xv.shape[0]
    bits = jax.lax.bitcast_convert_type(xv, jnp.int32) & jnp.int32(0x7FFFFFFF)
    hi = (bits >> 15).astype(jnp.int16) ^ jnp.int16(-0x8000)
    lo = (bits & 0x7FFF).astype(jnp.int16)
    kvec = jnp.full((rows, 1), k, jnp.int32)

    vstar, gt_v, _ = _value_search(hi, lo, kvec)

    r = kvec - gt_v
    idx_i = jax.lax.broadcasted_iota(jnp.int32, (rows, n_cols), 1)
    idx_f = idx_i.astype(jnp.float32)
    midx_v = jnp.where(bits == vstar, idx_f, jnp.float32(n_cols))
    t = jax.lax.cond(
        jnp.any(r > 1),
        lambda: _index_search(midx_v, r),
        lambda: _fold_min(midx_v).astype(jnp.int32),
    )

    ustar, _, ge_u = _value_search(hi, lo, jnp.int32(n_cols) - t)
    lcnt = jnp.int32(n_cols) - ge_u
    rp = t - lcnt
    match_u = bits == ustar
    midx_u = jnp.where(match_u, idx_f, jnp.float32(n_cols))
    j_cut = jax.lax.cond(
        jnp.any(rp > 1),
        lambda: _index_search(midx_u, rp),
        lambda: _fold_min(midx_u).astype(jnp.int32),
    )

    ustar_f = jax.lax.bitcast_convert_type(ustar, jnp.float32)
    keep = (jnp.abs(xv) < ustar_f) | (match_u & (idx_i <= j_cut) & (rp >= 1))
    o_ref[...] = xv * keep.astype(jnp.float32)


def kernel(x):
    n, c, h, w = x.shape
    n_cols = h * w
    k = int(max(1, _TOPK_FRAC * h * w))
    rows = n * c
    block_rows = 32
    while rows % block_rows:
        block_rows //= 2
    xr = x.reshape(rows, n_cols)

    out = pl.pallas_call(
        functools.partial(_body, k=k, n_cols=n_cols),
        grid=(rows // block_rows,),
        in_specs=[pl.BlockSpec((block_rows, n_cols), lambda i: (i, 0))],
        out_specs=pl.BlockSpec((block_rows, n_cols), lambda i: (i, 0)),
        out_shape=jax.ShapeDtypeStruct((rows, n_cols), jnp.float32),
    )(xr)
    return out.reshape(n, c, h, w)


# speedup vs baseline: 1.0318x; 1.0318x over previous
import functools

import jax
import jax.numpy as jnp
from jax.experimental import pallas as pl

_TOPK_FRAC = 0.1


def _psum(m, part_dtype):
    rows, n = m.shape
    mm = m.astype(part_dtype)
    if n % 128 == 0 and n > 128:
        acc = None
        while mm.shape[1] > 128:
            cols = mm.shape[1]
            if (cols // 128) % 2:
                tail = mm[:, cols - 128 :]
                acc = tail if acc is None else acc + tail
                mm = mm[:, : cols - 128]
            else:
                half = cols // 2
                mm = mm[:, :half] + mm[:, half:]
        if acc is not None:
            mm = mm + acc
    return jnp.sum(mm.astype(jnp.int32), axis=1, keepdims=True)


def _count16(plane, thr, strict=False):
    rows, n = plane.shape
    ways = 8 if n % 1024 == 0 and n > 1024 else 1
    q = n // ways
    s = None
    for j in range(ways):
        sl = plane[:, j * q : (j + 1) * q]
        m = (sl > thr) if strict else (sl >= thr)
        mj = m.astype(jnp.int16)
        s = mj if s is None else s + mj
    acc = None
    while s.shape[1] > 128:
        cols = s.shape[1]
        if (cols // 128) % 2:
            tail = s[:, cols - 128 :]
            acc = tail if acc is None else acc + tail
            s = s[:, : cols - 128]
        else:
            half = cols // 2
            s = s[:, :half] + s[:, half:]
    if acc is not None:
        s = s + acc
    return jnp.sum(s.astype(jnp.int32), axis=1, keepdims=True)


def _value_search(hi, lo, want):
    shape = want.shape

    def it_a(i, p):
        cand_u = p | ((1 << 15) >> i)
        cand = cand_u.astype(jnp.int16) ^ jnp.int16(-0x8000)
        ge = _count16(hi, cand)
        return jnp.where(ge >= want, cand_u, p)

    p_hi = jax.lax.fori_loop(0, 16, it_a, jnp.zeros(shape, jnp.int32))
    p_hi16 = p_hi.astype(jnp.int16) ^ jnp.int16(-0x8000)
    g1 = _count16(hi, p_hi16, strict=True)

    alo = jnp.where(hi == p_hi16, lo, jnp.int16(-1))
    want2 = want - g1

    def it_b(i, p):
        cand = (p | ((1 << 14) >> i)).astype(jnp.int16)
        ge = _count16(alo, cand)
        return jnp.where(ge >= want2, cand.astype(jnp.int32), p)

    p_lo = jax.lax.fori_loop(0, 15, it_b, jnp.zeros(shape, jnp.int32))
    p_lo16 = p_lo.astype(jnp.int16)
    g2 = _count16(alo, p_lo16, strict=True)
    ge2 = _count16(alo, p_lo16)

    vstar = (p_hi << 15) | p_lo
    return vstar, g1 + g2, g1 + ge2


def _fold_min(m):
    acc = None
    while m.shape[1] > 128:
        cols = m.shape[1]
        if (cols // 128) % 2 and cols > 256:
            tail = m[:, cols - 128 :]
            acc = tail if acc is None else jnp.minimum(acc, tail)
            m = m[:, : cols - 128]
        else:
            half = cols // 2
            m = jnp.minimum(m[:, :half], m[:, half:])
    if acc is not None:
        m = jnp.minimum(m, acc)
    return jnp.min(m, axis=1, keepdims=True)


def _index_search(midx, want):
    shape = want.shape

    def it(i, p):
        cand = p | ((1 << 15) >> i)
        c = _psum(midx < cand.astype(jnp.float32), jnp.float32)
        return jnp.where(c < want, cand, p)

    return jax.lax.fori_loop(0, 16, it, jnp.zeros(shape, jnp.int32))


def _body(x_ref, o_ref, *, k, n_cols):
    xv = x_ref[...]
    rows = xv.shape[0]
    bits = jax.lax.bitcast_convert_type(xv, jnp.int32) & jnp.int32(0x7FFFFFFF)
    hi = (bits >> 15).astype(jnp.int16) ^ jnp.int16(-0x8000)
    lo = (bits & 0x7FFF).astype(jnp.int16)
    kvec = jnp.full((rows, 1), k, jnp.int32)

    vstar, gt_v, _ = _value_search(hi, lo, kvec)

    r = kvec - gt_v
    idx_i = jax.lax.broadcasted_iota(jnp.int32, (rows, n_cols), 1)
    idx_f = idx_i.astype(jnp.float32)
    midx_v = jnp.where(bits == vstar, idx_f, jnp.float32(n_cols))
    t = jax.lax.cond(
        jnp.any(r > 1),
        lambda: _index_search(midx_v, r),
        lambda: _fold_min(midx_v).astype(jnp.int32),
    )

    ustar, _, ge_u = _value_search(hi, lo, jnp.int32(n_cols) - t)
    lcnt = jnp.int32(n_cols) - ge_u
    rp = t - lcnt
    match_u = bits == ustar
    midx_u = jnp.where(match_u, idx_f, jnp.float32(n_cols))
    j_cut = jax.lax.cond(
        jnp.any(rp > 1),
        lambda: _index_search(midx_u, rp),
        lambda: _fold_min(midx_u).astype(jnp.int32),
    )

    ustar_f = jax.lax.bitcast_convert_type(ustar, jnp.float32)
    keep = (jnp.abs(xv) < ustar_f) | (match_u & (idx_i <= j_cut) & (rp >= 1))
    o_ref[...] = xv * keep.astype(jnp.float32)


def kernel(x):
    n, c, h, w = x.shape
    n_cols = h * w
    k = int(max(1, _TOPK_FRAC * h * w))
    rows = n * c
    block_rows = 32
    while rows % block_rows:
        block_rows //= 2
    xr = x.reshape(rows, n_cols)

    out = pl.pallas_call(
        functools.partial(_body, k=k, n_cols=n_cols),
        grid=(rows // block_rows,),
        in_specs=[pl.BlockSpec((block_rows, n_cols), lambda i: (i, 0))],
        out_specs=pl.BlockSpec((block_rows, n_cols), lambda i: (i, 0)),
        out_shape=jax.ShapeDtypeStruct((rows, n_cols), jnp.float32),
    )(xr)
    return out.reshape(n, c, h, w)
